# baseline (speedup 1.0000x reference)
import functools

import jax
import jax.numpy as jnp
from jax import lax
from jax.experimental import pallas as pl
from jax.experimental.pallas import tpu as pltpu
from jax.experimental.pallas import tpu_sc as plsc

_B = 2
_D = 128
_NR = 3
_NC = 2
_NS = 16
_K = 128


def _pad_up(n, m):
    return ((n + m - 1) // m) * m


@functools.lru_cache(maxsize=None)
def _make_spmm(N_p, n_chunks):
    rows_pc = N_p // _NS
    mesh = plsc.VectorSubcoreMesh(core_axis_name="c", subcore_axis_name="s",
                                  num_cores=_NC, num_subcores=_NS)

    @functools.partial(
        pl.kernel,
        out_type=jax.ShapeDtypeStruct((_B, _NR, N_p, _D), jnp.float32),
        mesh=mesh,
        scratch_types=[
            pltpu.VMEM_SHARED((N_p, _D), jnp.float32),
            pltpu.VMEM((_K,), jnp.int32),
            pltpu.VMEM((_K,), jnp.int32),
            pltpu.VMEM((_K, _D), jnp.float32),
            pltpu.VMEM((8, _D), jnp.float32),
            pltpu.VMEM((8, _D), jnp.float32),
            pltpu.SemaphoreType.DMA,
        ],
    )
    def spmm(tables, srcp, dstp, out, acc, idx_s, idx_d, rows, zbuf, obuf,
             gsem):
        sid = lax.axis_index("s")
        b = lax.axis_index("c")
        row0 = sid * rows_pc
        zero16 = jnp.zeros((16,), jnp.float32)
        for rr in range(8):
            for cc in range(_D // 16):
                zbuf[rr, pl.ds(cc * 16, 16)] = zero16
        for r in range(_NR):
            def zbody(j, c):
                pltpu.sync_copy(zbuf, acc.at[pl.ds(row0 + j * 8, 8)])
                return c
            lax.fori_loop(0, rows_pc // 8, zbody, 0)
            plsc.subcore_barrier()

            def ebody(k, c):
                pltpu.sync_copy(srcp.at[b, r, sid, pl.ds(k * _K, _K)], idx_s)
                pltpu.sync_copy(dstp.at[r, sid, pl.ds(k * _K, _K)], idx_d)
                pltpu.async_copy(tables.at[idx_s], rows, gsem).wait()
                pltpu.sync_copy(rows, acc.at[idx_d], add=True)
                return c
            lax.fori_loop(0, n_chunks, ebody, 0)
            plsc.subcore_barrier()

            def obody(j, c):
                pltpu.sync_copy(acc.at[pl.ds(row0 + j * 8, 8)], obuf)
                pltpu.sync_copy(obuf, out.at[b, r, pl.ds(row0 + j * 8, 8)])
                return c
            lax.fori_loop(0, rows_pc // 8, obody, 0)
            plsc.subcore_barrier()

    return spmm


@functools.lru_cache(maxsize=None)
def _make_degrees(N_p, n_chunks):
    rows_pc = N_p // _NS
    mesh = plsc.VectorSubcoreMesh(core_axis_name="c", subcore_axis_name="s",
                                  num_cores=_NC, num_subcores=_NS)

    @functools.partial(
        pl.kernel,
        out_type=jax.ShapeDtypeStruct((2 * _NR, N_p, _D), jnp.float32),
        mesh=mesh,
        scratch_types=[
            pltpu.VMEM_SHARED((N_p, _D), jnp.float32),
            pltpu.VMEM((_K,), jnp.int32),
            pltpu.VMEM((_K, _D), jnp.float32),
            pltpu.VMEM((8, _D), jnp.float32),
            pltpu.VMEM((8, _D), jnp.float32),
        ],
    )
    def deg(ones_tbl, edges_q, out, acc, idxb, ones, zb, ob):
        sid = lax.axis_index("s")
        b = lax.axis_index("c")
        row0 = sid * rows_pc
        zero16 = jnp.zeros((16,), jnp.float32)
        for rr in range(8):
            for cc in range(_D // 16):
                zb[rr, pl.ds(cc * 16, 16)] = zero16
        pltpu.sync_copy(ones_tbl, ones)
        for j3 in range(_NR):
            q = b * _NR + j3
            def zbody(j, c):
                pltpu.sync_copy(zb, acc.at[pl.ds(row0 + j * 8, 8)])
                return c
            lax.fori_loop(0, rows_pc // 8, zbody, 0)
            plsc.subcore_barrier()

            def ebody(k, c):
                pltpu.sync_copy(edges_q.at[q, sid, pl.ds(k * _K, _K)], idxb)
                pltpu.sync_copy(ones, acc.at[idxb], add=True)
                return c
            lax.fori_loop(0, n_chunks, ebody, 0)
            plsc.subcore_barrier()

            def obody(j, c):
                pltpu.sync_copy(acc.at[pl.ds(row0 + j * 8, 8)], ob)
                pltpu.sync_copy(ob, out.at[q, pl.ds(row0 + j * 8, 8)])
                return c
            lax.fori_loop(0, rows_pc // 8, obody, 0)
            plsc.subcore_barrier()

    return deg


def _mm_scale(x, W3, cnt_src):
    B, N_p, D = x.shape
    nblk = N_p // _K

    def body(x_ref, w_ref, c_ref, o_ref):
        do = lax.rsqrt(jnp.maximum(c_ref[0, :, 0:1], 1.0))
        o_ref[0, 0] = jnp.dot(x_ref[0], w_ref[0],
                              preferred_element_type=jnp.float32) * do

    return pl.pallas_call(
        body,
        grid=(B, _NR, nblk),
        in_specs=[
            pl.BlockSpec((1, _K, D), lambda b, r, i: (b, i, 0)),
            pl.BlockSpec((1, D, D), lambda b, r, i: (r, 0, 0)),
            pl.BlockSpec((1, _K, D), lambda b, r, i: (r, i, 0)),
        ],
        out_specs=pl.BlockSpec((1, 1, _K, D), lambda b, r, i: (b, r, i, 0)),
        out_shape=jax.ShapeDtypeStruct((B, _NR, N_p, D), jnp.float32),
    )(x, W3, cnt_src)


def _combine(agg, cnt_dst, aw):
    B, NR, N_p, D = agg.shape
    nblk = N_p // _K

    def body(a_ref, c_ref, aw_ref, o_ref):
        b = pl.program_id(0)
        acc = jnp.zeros((_K, D), jnp.float32)
        for r in range(_NR):
            di = lax.rsqrt(jnp.maximum(c_ref[r, :, 0:1], 1.0))
            acc = acc + jax.nn.relu(a_ref[0, r] * (di * aw_ref[b, r]))
        o_ref[0] = acc

    return pl.pallas_call(
        body,
        grid=(B, nblk),
        in_specs=[
            pl.BlockSpec((1, _NR, _K, D), lambda b, i: (b, 0, i, 0)),
            pl.BlockSpec((_NR, _K, D), lambda b, i: (0, i, 0)),
            pl.BlockSpec(memory_space=pltpu.SMEM),
        ],
        out_specs=pl.BlockSpec((1, _K, D), lambda b, i: (b, i, 0)),
        out_shape=jax.ShapeDtypeStruct((B, N_p, D), jnp.float32),
    )(agg, cnt_dst, aw)


def _boundary(cagg, cnt_dst, aw, i_feat, W_proj, b_proj2, W_conf, b_conf2,
              W_lin, tgt, N1):
    B, NR, N2p, D = cagg.shape
    N1p = i_feat.shape[1]
    nblk = N1p // _K

    def body(a_ref, c_ref, aw_ref, if_ref, wp_ref, bp_ref, wc_ref, bc_ref,
             wl_ref, t_ref, f_ref, ct_ref, p_ref, n_ref):
        b = pl.program_id(0)
        i = pl.program_id(1)
        acc = jnp.zeros((_K, D), jnp.float32)
        for r in range(_NR):
            di = lax.rsqrt(jnp.maximum(c_ref[r, :, 0:1], 1.0))
            acc = acc + jax.nn.relu(a_ref[0, r] * (di * aw_ref[b, r]))
        proj = jnp.dot(acc, wp_ref[...], preferred_element_type=jnp.float32)
        proj = proj + bp_ref[0]
        conf = jax.nn.sigmoid(
            jnp.dot(acc, wc_ref[...], preferred_element_type=jnp.float32)
            + bc_ref[0])
        f = conf * (if_ref[0] + proj)
        rows = i * _K + lax.broadcasted_iota(jnp.int32, (_K, 1), 0)
        f = f * (rows < N1).astype(jnp.float32)
        f_ref[0] = f
        ct_ref[0] = jnp.dot(f, wl_ref[...], preferred_element_type=jnp.float32)
        psel = jnp.sum(jnp.where(rows == t_ref[b, 0], f, 0.0), axis=0)
        nsel = jnp.sum(jnp.where(rows == t_ref[b, 1], f, 0.0), axis=0)
        psel = jnp.broadcast_to(psel[None, :], (8, f.shape[1]))
        nsel = jnp.broadcast_to(nsel[None, :], (8, f.shape[1]))
        first = (i == 0)
        p_ref[0] = jnp.where(first, psel, p_ref[0] + psel)
        n_ref[0] = jnp.where(first, nsel, n_ref[0] + nsel)

    return pl.pallas_call(
        body,
        grid=(B, nblk),
        in_specs=[
            pl.BlockSpec((1, _NR, _K, D), lambda b, i: (b, 0, i, 0)),
            pl.BlockSpec((_NR, _K, D), lambda b, i: (0, i, 0)),
            pl.BlockSpec(memory_space=pltpu.SMEM),
            pl.BlockSpec((1, _K, D), lambda b, i: (b, i, 0)),
            pl.BlockSpec((D, D), lambda b, i: (0, 0)),
            pl.BlockSpec((1, D), lambda b, i: (0, 0)),
            pl.BlockSpec((D, D), lambda b, i: (0, 0)),
            pl.BlockSpec((1, D), lambda b, i: (0, 0)),
            pl.BlockSpec((D, D), lambda b, i: (0, 0)),
            pl.BlockSpec(memory_space=pltpu.SMEM),
        ],
        out_specs=[
            pl.BlockSpec((1, _K, D), lambda b, i: (b, i, 0)),
            pl.BlockSpec((1, _K, D), lambda b, i: (b, i, 0)),
            pl.BlockSpec((1, 8, D), lambda b, i: (b, 0, 0)),
            pl.BlockSpec((1, 8, D), lambda b, i: (b, 0, 0)),
        ],
        out_shape=[
            jax.ShapeDtypeStruct((B, N1p, D), jnp.float32),
            jax.ShapeDtypeStruct((B, N1p, D), jnp.float32),
            jax.ShapeDtypeStruct((B, 8, D), jnp.float32),
            jax.ShapeDtypeStruct((B, 8, D), jnp.float32),
        ],
    )(cagg, cnt_dst, aw, i_feat, W_proj, b_proj2, W_conf, b_conf2, W_lin, tgt)


def _seed(hof3, tm, herb3, W_lin):
    B = hof3.shape[0]
    NHp = hof3.shape[2]
    N1p = tm.shape[1]
    D = herb3.shape[2]
    nblk = N1p // _K

    def body(m_ref, tm_ref, h_ref, wl_ref, i0_ref, c0_ref):
        m = (m_ref[0] > 0).astype(jnp.float32)
        s = jnp.dot(m, tm_ref[...], preferred_element_type=jnp.float32)
        h = h_ref[0]
        hw = jnp.dot(h, wl_ref[...], preferred_element_type=jnp.float32)
        i0_ref[0] = s[0][:, None] * h[0][None, :]
        c0_ref[0] = s[0][:, None] * hw[0][None, :]

    return pl.pallas_call(
        body,
        grid=(B, nblk),
        in_specs=[
            pl.BlockSpec((1, 1, NHp), lambda b, i: (b, 0, 0)),
            pl.BlockSpec((NHp, _K), lambda b, i: (0, i)),
            pl.BlockSpec((1, 1, D), lambda b, i: (b, 0, 0)),
            pl.BlockSpec((D, D), lambda b, i: (0, 0)),
        ],
        out_specs=[
            pl.BlockSpec((1, _K, D), lambda b, i: (b, i, 0)),
            pl.BlockSpec((1, _K, D), lambda b, i: (b, i, 0)),
        ],
        out_shape=[
            jax.ShapeDtypeStruct((B, N1p, D), jnp.float32),
            jax.ShapeDtypeStruct((B, N1p, D), jnp.float32),
        ],
    )(hof3, tm, herb3, W_lin)


def _attn(herb, Wcat, bcat):
    B, D = herb.shape
    M = Wcat.shape[1]

    def body(x_ref, w_ref, b_ref, o_ref):
        o_ref[...] = jax.nn.sigmoid(
            jnp.dot(x_ref[...], w_ref[...],
                    preferred_element_type=jnp.float32) + b_ref[0])

    return pl.pallas_call(
        body,
        grid=(1,),
        in_specs=[
            pl.BlockSpec((B, D), lambda k: (0, 0)),
            pl.BlockSpec((D, M), lambda k: (0, 0)),
            pl.BlockSpec((1, M), lambda k: (0, 0)),
        ],
        out_specs=pl.BlockSpec((B, M), lambda k: (0, 0)),
        out_shape=jax.ShapeDtypeStruct((B, M), jnp.float32),
    )(herb, Wcat, bcat)


def _prep_edges(edges, N, N_p):
    E = edges.shape[2]
    epc = E // _NS
    epc_p = _pad_up(epc, _K)
    n_chunks = epc_p // _K
    pad = epc_p - epc
    e = edges.astype(jnp.int32).reshape(_NR, 2, _NS, epc)
    src = jnp.pad(e[:, 0], ((0, 0), (0, 0), (0, pad)))
    dst = jnp.pad(e[:, 1], ((0, 0), (0, 0), (0, pad)),
                  constant_values=N)
    roff = (jnp.arange(_NR, dtype=jnp.int32) * N_p)[None, :, None, None]
    boff = (jnp.arange(_B, dtype=jnp.int32) * (_NR * N_p))[:, None, None, None]
    srcp = src[None] + roff + boff
    src_q = jnp.pad(e[:, 0], ((0, 0), (0, 0), (0, pad)), constant_values=N)
    edges_q = jnp.stack([src_q, dst], axis=1).reshape(2 * _NR, _NS, epc_p)
    return srcp, dst, edges_q, n_chunks


def _counts(edges_q, N_p, n_chunks):
    ones_tbl = jnp.ones((_K, _D), jnp.float32)
    cnt = _make_degrees(N_p, n_chunks)(ones_tbl, edges_q)
    return cnt[0::2], cnt[1::2]


def _conv(x, W3, aw, cs, cd, srcp, dstp, N_p, n_chunks):
    y = _mm_scale(x, W3, cs)
    agg = _make_spmm(N_p, n_chunks)(y.reshape(_B * _NR * N_p, _D), srcp, dstp)
    return _combine(agg, cd, aw)


def kernel(herb_feature, herb_original_feat, graph1_edges, graph2_edges,
           p_targets, n_targets, total_map, W_lin, Wa1, ba1, Wa2, ba2,
           Wc1, Wc2, W_proj, b_proj, W_conf, b_conf):
    B, NH = herb_original_feat.shape
    N1 = total_map.shape[1]
    N2 = int(graph2_edges.shape[2] // 16)
    LI_LO = Wa1.shape[0]
    LO = Wa2.shape[0]
    LI = LI_LO // LO
    NR = Wa1.shape[2]
    N1p = _pad_up(N1, _NS * 8)
    N2p = _pad_up(N2, _NS * 8)
    NHp = _pad_up(NH, 512)

    src1, dst1, eq1, nc1 = _prep_edges(graph1_edges, N1, N1p)
    src2, dst2, eq2, nc2 = _prep_edges(graph2_edges, N2, N2p)
    cs1, cd1 = _counts(eq1, N1p, nc1)
    cs2, cd2 = _counts(eq2, N2p, nc2)

    hof3 = jnp.pad(herb_original_feat.astype(jnp.int32),
                   ((0, 0), (0, NHp - NH)))[:, None, :]
    tmp = jnp.pad(total_map, ((0, NHp - NH), (0, N1p - N1)))
    herb3 = herb_feature[:, None, :]
    i_feat, ctop = _seed(hof3, tmp, herb3, W_lin)

    Wcat = jnp.concatenate(list(Wa1) + list(Wa2), axis=1)
    bcat = jnp.concatenate(list(ba1) + list(ba2), axis=0)[None, :]
    M = _pad_up(Wcat.shape[1], 128)
    Wcat = jnp.pad(Wcat, ((0, 0), (0, M - Wcat.shape[1])))
    bcat = jnp.pad(bcat, ((0, 0), (0, M - bcat.shape[1])))
    aw = _attn(herb_feature, Wcat, bcat)

    tgt = jnp.concatenate([p_targets.reshape(-1, 1),
                           n_targets.reshape(-1, 1)], axis=1).astype(jnp.int32)
    b_proj2 = b_proj[None, :]
    b_conf2 = b_conf[None, :]

    c_feat = jnp.pad(ctop, ((0, 0), (0, N2p - N1p), (0, 0)))
    p_feat = n_feat = None
    for o in range(LO):
        for i in range(LI):
            idx = o * LI + i
            aw1 = aw[:, idx * NR:(idx + 1) * NR]
            i_feat = _conv(i_feat, Wc1[idx], aw1, cs1, cd1,
                           src1, dst1, N1p, nc1)
        aw2 = aw[:, (LI_LO + o) * NR:(LI_LO + o + 1) * NR]
        c_feat = _conv(c_feat, Wc2[o], aw2, cs2, cd2, src2, dst2, N2p, nc2)
        y2 = _mm_scale(c_feat, Wc2[o], cs2)
        cagg = _make_spmm(N2p, nc2)(y2.reshape(_B * _NR * N2p, _D),
                                    src2, dst2)
        i_feat, ctop, p_feat, n_feat = _boundary(
            cagg, cd2, aw2, i_feat, W_proj, b_proj2, W_conf, b_conf2,
            W_lin, tgt, N1)
        if o + 1 < LO:
            c_feat = jnp.pad(ctop, ((0, 0), (0, N2p - N1p), (0, 0)))

    return (p_feat[:, 0, :], n_feat[:, 0, :])


# speedup vs baseline: 1.0681x; 1.0681x over previous
import functools

import jax
import jax.numpy as jnp
from jax import lax
from jax.experimental import pallas as pl
from jax.experimental.pallas import tpu as pltpu
from jax.experimental.pallas import tpu_sc as plsc

_B = 2
_D = 128
_NR = 3
_NC = 2
_NS = 16
_K = 128


def _pad_up(n, m):
    return ((n + m - 1) // m) * m


@functools.lru_cache(maxsize=None)
def _make_spmm(N_p, n_chunks, kb):
    rows_pc = N_p // _NS
    mesh = plsc.VectorSubcoreMesh(core_axis_name="c", subcore_axis_name="s",
                                  num_cores=_NC, num_subcores=_NS)

    @functools.partial(
        pl.kernel,
        out_type=jax.ShapeDtypeStruct((_B, _NR, N_p, _D), jnp.float32),
        mesh=mesh,
        scratch_types=[
            pltpu.VMEM_SHARED((N_p, _D), jnp.float32),
            pltpu.VMEM((kb,), jnp.int32),
            pltpu.VMEM((kb,), jnp.int32),
            pltpu.VMEM((kb,), jnp.int32),
            pltpu.VMEM((kb,), jnp.int32),
            pltpu.VMEM((kb, _D), jnp.float32),
            pltpu.VMEM((kb, _D), jnp.float32),
            pltpu.VMEM((8, _D), jnp.float32),
            pltpu.VMEM((8, _D), jnp.float32),
            pltpu.SemaphoreType.DMA,
            pltpu.SemaphoreType.DMA,
        ],
    )
    def spmm(tables, srcp, dstp, out, acc, s0, s1, d0, d1, rows0, rows1,
             zbuf, obuf, sem0, sem1):
        sid = lax.axis_index("s")
        b = lax.axis_index("c")
        row0 = sid * rows_pc
        zero16 = jnp.zeros((16,), jnp.float32)
        for rr in range(8):
            for cc in range(_D // 16):
                zbuf[rr, pl.ds(cc * 16, 16)] = zero16
        for r in range(_NR):
            def zbody(j, c):
                pltpu.sync_copy(zbuf, acc.at[pl.ds(row0 + j * 8, 8)])
                return c
            lax.fori_loop(0, rows_pc // 8, zbody, 0)
            plsc.subcore_barrier()

            def ebody(k2, c):
                pltpu.sync_copy(srcp.at[b, r, sid, 2 * k2], s0)
                pltpu.sync_copy(srcp.at[b, r, sid, 2 * k2 + 1], s1)
                g0 = pltpu.async_copy(tables.at[s0], rows0, sem0)
                g1 = pltpu.async_copy(tables.at[s1], rows1, sem1)
                pltpu.sync_copy(dstp.at[r, sid, 2 * k2], d0)
                pltpu.sync_copy(dstp.at[r, sid, 2 * k2 + 1], d1)
                g0.wait()
                pltpu.sync_copy(rows0, acc.at[d0], add=True)
                g1.wait()
                pltpu.sync_copy(rows1, acc.at[d1], add=True)
                return c
            lax.fori_loop(0, n_chunks // 2, ebody, 0)
            plsc.subcore_barrier()

            def obody(j, c):
                pltpu.sync_copy(acc.at[pl.ds(row0 + j * 8, 8)], obuf)
                pltpu.sync_copy(obuf, out.at[b, r, pl.ds(row0 + j * 8, 8)])
                return c
            lax.fori_loop(0, rows_pc // 8, obody, 0)
            plsc.subcore_barrier()

    return spmm


@functools.lru_cache(maxsize=None)
def _make_degrees(N_p, n_chunks):
    rows_pc = N_p // _NS
    mesh = plsc.VectorSubcoreMesh(core_axis_name="c", subcore_axis_name="s",
                                  num_cores=_NC, num_subcores=_NS)

    @functools.partial(
        pl.kernel,
        out_type=jax.ShapeDtypeStruct((2 * _NR, N_p, _D), jnp.float32),
        mesh=mesh,
        scratch_types=[
            pltpu.VMEM_SHARED((N_p, _D), jnp.float32),
            pltpu.VMEM((_K,), jnp.int32),
            pltpu.VMEM((_K, _D), jnp.float32),
            pltpu.VMEM((8, _D), jnp.float32),
            pltpu.VMEM((8, _D), jnp.float32),
        ],
    )
    def deg(ones_tbl, edges_q, out, acc, idxb, ones, zb, ob):
        sid = lax.axis_index("s")
        b = lax.axis_index("c")
        row0 = sid * rows_pc
        zero16 = jnp.zeros((16,), jnp.float32)
        for rr in range(8):
            for cc in range(_D // 16):
                zb[rr, pl.ds(cc * 16, 16)] = zero16
        pltpu.sync_copy(ones_tbl, ones)
        for j3 in range(_NR):
            q = b * _NR + j3
            def zbody(j, c):
                pltpu.sync_copy(zb, acc.at[pl.ds(row0 + j * 8, 8)])
                return c
            lax.fori_loop(0, rows_pc // 8, zbody, 0)
            plsc.subcore_barrier()

            def ebody(k, c):
                pltpu.sync_copy(edges_q.at[q, sid, pl.ds(k * _K, _K)], idxb)
                pltpu.sync_copy(ones, acc.at[idxb], add=True)
                return c
            lax.fori_loop(0, n_chunks, ebody, 0)
            plsc.subcore_barrier()

            def obody(j, c):
                pltpu.sync_copy(acc.at[pl.ds(row0 + j * 8, 8)], ob)
                pltpu.sync_copy(ob, out.at[q, pl.ds(row0 + j * 8, 8)])
                return c
            lax.fori_loop(0, rows_pc // 8, obody, 0)
            plsc.subcore_barrier()

    return deg


def _mm_scale(x, W3, cnt_src):
    B, N_p, D = x.shape
    nblk = N_p // _K

    def body(x_ref, w_ref, c_ref, o_ref):
        do = lax.rsqrt(jnp.maximum(c_ref[0, :, 0:1], 1.0))
        o_ref[0, 0] = jnp.dot(x_ref[0], w_ref[0],
                              preferred_element_type=jnp.float32) * do

    return pl.pallas_call(
        body,
        grid=(B, _NR, nblk),
        in_specs=[
            pl.BlockSpec((1, _K, D), lambda b, r, i: (b, i, 0)),
            pl.BlockSpec((1, D, D), lambda b, r, i: (r, 0, 0)),
            pl.BlockSpec((1, _K, D), lambda b, r, i: (r, i, 0)),
        ],
        out_specs=pl.BlockSpec((1, 1, _K, D), lambda b, r, i: (b, r, i, 0)),
        out_shape=jax.ShapeDtypeStruct((B, _NR, N_p, D), jnp.float32),
    )(x, W3, cnt_src)


def _combine(agg, cnt_dst, aw):
    B, NR, N_p, D = agg.shape
    nblk = N_p // _K

    def body(a_ref, c_ref, aw_ref, o_ref):
        b = pl.program_id(0)
        acc = jnp.zeros((_K, D), jnp.float32)
        for r in range(_NR):
            di = lax.rsqrt(jnp.maximum(c_ref[r, :, 0:1], 1.0))
            acc = acc + jax.nn.relu(a_ref[0, r] * (di * aw_ref[b, r]))
        o_ref[0] = acc

    return pl.pallas_call(
        body,
        grid=(B, nblk),
        in_specs=[
            pl.BlockSpec((1, _NR, _K, D), lambda b, i: (b, 0, i, 0)),
            pl.BlockSpec((_NR, _K, D), lambda b, i: (0, i, 0)),
            pl.BlockSpec(memory_space=pltpu.SMEM),
        ],
        out_specs=pl.BlockSpec((1, _K, D), lambda b, i: (b, i, 0)),
        out_shape=jax.ShapeDtypeStruct((B, N_p, D), jnp.float32),
    )(agg, cnt_dst, aw)


def _boundary(cagg, cnt_dst, aw, i_feat, W_proj, b_proj2, W_conf, b_conf2,
              W_lin, tgt, N1):
    B, NR, N2p, D = cagg.shape
    N1p = i_feat.shape[1]
    nblk = N1p // _K

    def body(a_ref, c_ref, aw_ref, if_ref, wp_ref, bp_ref, wc_ref, bc_ref,
             wl_ref, t_ref, f_ref, ct_ref, p_ref, n_ref):
        b = pl.program_id(0)
        i = pl.program_id(1)
        acc = jnp.zeros((_K, D), jnp.float32)
        for r in range(_NR):
            di = lax.rsqrt(jnp.maximum(c_ref[r, :, 0:1], 1.0))
            acc = acc + jax.nn.relu(a_ref[0, r] * (di * aw_ref[b, r]))
        proj = jnp.dot(acc, wp_ref[...], preferred_element_type=jnp.float32)
        proj = proj + bp_ref[0]
        conf = jax.nn.sigmoid(
            jnp.dot(acc, wc_ref[...], preferred_element_type=jnp.float32)
            + bc_ref[0])
        f = conf * (if_ref[0] + proj)
        rows = i * _K + lax.broadcasted_iota(jnp.int32, (_K, 1), 0)
        f = f * (rows < N1).astype(jnp.float32)
        f_ref[0] = f
        ct_ref[0] = jnp.dot(f, wl_ref[...], preferred_element_type=jnp.float32)
        psel = jnp.sum(jnp.where(rows == t_ref[b, 0], f, 0.0), axis=0)
        nsel = jnp.sum(jnp.where(rows == t_ref[b, 1], f, 0.0), axis=0)
        psel = jnp.broadcast_to(psel[None, :], (8, f.shape[1]))
        nsel = jnp.broadcast_to(nsel[None, :], (8, f.shape[1]))
        first = (i == 0)
        p_ref[0] = jnp.where(first, psel, p_ref[0] + psel)
        n_ref[0] = jnp.where(first, nsel, n_ref[0] + nsel)

    return pl.pallas_call(
        body,
        grid=(B, nblk),
        in_specs=[
            pl.BlockSpec((1, _NR, _K, D), lambda b, i: (b, 0, i, 0)),
            pl.BlockSpec((_NR, _K, D), lambda b, i: (0, i, 0)),
            pl.BlockSpec(memory_space=pltpu.SMEM),
            pl.BlockSpec((1, _K, D), lambda b, i: (b, i, 0)),
            pl.BlockSpec((D, D), lambda b, i: (0, 0)),
            pl.BlockSpec((1, D), lambda b, i: (0, 0)),
            pl.BlockSpec((D, D), lambda b, i: (0, 0)),
            pl.BlockSpec((1, D), lambda b, i: (0, 0)),
            pl.BlockSpec((D, D), lambda b, i: (0, 0)),
            pl.BlockSpec(memory_space=pltpu.SMEM),
        ],
        out_specs=[
            pl.BlockSpec((1, _K, D), lambda b, i: (b, i, 0)),
            pl.BlockSpec((1, _K, D), lambda b, i: (b, i, 0)),
            pl.BlockSpec((1, 8, D), lambda b, i: (b, 0, 0)),
            pl.BlockSpec((1, 8, D), lambda b, i: (b, 0, 0)),
        ],
        out_shape=[
            jax.ShapeDtypeStruct((B, N1p, D), jnp.float32),
            jax.ShapeDtypeStruct((B, N1p, D), jnp.float32),
            jax.ShapeDtypeStruct((B, 8, D), jnp.float32),
            jax.ShapeDtypeStruct((B, 8, D), jnp.float32),
        ],
    )(cagg, cnt_dst, aw, i_feat, W_proj, b_proj2, W_conf, b_conf2, W_lin, tgt)


def _seed(hof3, tm, herb3, W_lin):
    B = hof3.shape[0]
    NHp = hof3.shape[2]
    N1p = tm.shape[1]
    D = herb3.shape[2]
    nblk = N1p // _K

    def body(m_ref, tm_ref, h_ref, wl_ref, i0_ref, c0_ref):
        m = (m_ref[0] > 0).astype(jnp.float32)
        s = jnp.dot(m, tm_ref[...], preferred_element_type=jnp.float32)
        h = h_ref[0]
        hw = jnp.dot(h, wl_ref[...], preferred_element_type=jnp.float32)
        i0_ref[0] = s[0][:, None] * h[0][None, :]
        c0_ref[0] = s[0][:, None] * hw[0][None, :]

    return pl.pallas_call(
        body,
        grid=(B, nblk),
        in_specs=[
            pl.BlockSpec((1, 1, NHp), lambda b, i: (b, 0, 0)),
            pl.BlockSpec((NHp, _K), lambda b, i: (0, i)),
            pl.BlockSpec((1, 1, D), lambda b, i: (b, 0, 0)),
            pl.BlockSpec((D, D), lambda b, i: (0, 0)),
        ],
        out_specs=[
            pl.BlockSpec((1, _K, D), lambda b, i: (b, i, 0)),
            pl.BlockSpec((1, _K, D), lambda b, i: (b, i, 0)),
        ],
        out_shape=[
            jax.ShapeDtypeStruct((B, N1p, D), jnp.float32),
            jax.ShapeDtypeStruct((B, N1p, D), jnp.float32),
        ],
    )(hof3, tm, herb3, W_lin)


def _attn(herb, Wcat, bcat):
    B, D = herb.shape
    M = Wcat.shape[1]

    def body(x_ref, w_ref, b_ref, o_ref):
        o_ref[...] = jax.nn.sigmoid(
            jnp.dot(x_ref[...], w_ref[...],
                    preferred_element_type=jnp.float32) + b_ref[0])

    return pl.pallas_call(
        body,
        grid=(1,),
        in_specs=[
            pl.BlockSpec((B, D), lambda k: (0, 0)),
            pl.BlockSpec((D, M), lambda k: (0, 0)),
            pl.BlockSpec((1, M), lambda k: (0, 0)),
        ],
        out_specs=pl.BlockSpec((B, M), lambda k: (0, 0)),
        out_shape=jax.ShapeDtypeStruct((B, M), jnp.float32),
    )(herb, Wcat, bcat)


def _prep_edges(edges, N, N_p, kb):
    E = edges.shape[2]
    epc = E // _NS
    epc_p = _pad_up(epc, max(2 * kb, _K))
    n_chunks = epc_p // kb
    pad = epc_p - epc
    e = edges.astype(jnp.int32).reshape(_NR, 2, _NS, epc)
    src = jnp.pad(e[:, 0], ((0, 0), (0, 0), (0, pad)))
    dst = jnp.pad(e[:, 1], ((0, 0), (0, 0), (0, pad)),
                  constant_values=N)
    roff = (jnp.arange(_NR, dtype=jnp.int32) * N_p)[None, :, None, None]
    boff = (jnp.arange(_B, dtype=jnp.int32) * (_NR * N_p))[:, None, None, None]
    srcp = (src[None] + roff + boff).reshape(_B, _NR, _NS, n_chunks, kb)
    dstp = dst.reshape(_NR, _NS, n_chunks, kb)
    src_q = jnp.pad(e[:, 0], ((0, 0), (0, 0), (0, pad)), constant_values=N)
    edges_q = jnp.stack([src_q, dst], axis=1).reshape(2 * _NR, _NS, epc_p)
    return srcp, dstp, edges_q, n_chunks, epc_p // _K


def _counts(edges_q, N_p, n_chunks):
    ones_tbl = jnp.ones((_K, _D), jnp.float32)
    cnt = _make_degrees(N_p, n_chunks)(ones_tbl, edges_q)
    return cnt[0::2], cnt[1::2]


def _conv(x, W3, aw, cs, cd, srcp, dstp, N_p, n_chunks, kb):
    y = _mm_scale(x, W3, cs)
    agg = _make_spmm(N_p, n_chunks, kb)(y.reshape(_B * _NR * N_p, _D),
                                        srcp, dstp)
    return _combine(agg, cd, aw)


def kernel(herb_feature, herb_original_feat, graph1_edges, graph2_edges,
           p_targets, n_targets, total_map, W_lin, Wa1, ba1, Wa2, ba2,
           Wc1, Wc2, W_proj, b_proj, W_conf, b_conf):
    B, NH = herb_original_feat.shape
    N1 = total_map.shape[1]
    N2 = int(graph2_edges.shape[2] // 16)
    LI_LO = Wa1.shape[0]
    LO = Wa2.shape[0]
    LI = LI_LO // LO
    NR = Wa1.shape[2]
    N1p = _pad_up(N1, _NS * 8)
    N2p = _pad_up(N2, _NS * 8)
    NHp = _pad_up(NH, 512)

    kb1 = 128 if N1p <= 8192 else 64
    kb2 = 128 if N2p <= 8192 else 64
    src1, dst1, eq1, nc1, ncq1 = _prep_edges(graph1_edges, N1, N1p, kb1)
    src2, dst2, eq2, nc2, ncq2 = _prep_edges(graph2_edges, N2, N2p, kb2)
    cs1, cd1 = _counts(eq1, N1p, ncq1)
    cs2, cd2 = _counts(eq2, N2p, ncq2)

    hof3 = jnp.pad(herb_original_feat.astype(jnp.int32),
                   ((0, 0), (0, NHp - NH)))[:, None, :]
    tmp = jnp.pad(total_map, ((0, NHp - NH), (0, N1p - N1)))
    herb3 = herb_feature[:, None, :]
    i_feat, ctop = _seed(hof3, tmp, herb3, W_lin)

    Wcat = jnp.concatenate(list(Wa1) + list(Wa2), axis=1)
    bcat = jnp.concatenate(list(ba1) + list(ba2), axis=0)[None, :]
    M = _pad_up(Wcat.shape[1], 128)
    Wcat = jnp.pad(Wcat, ((0, 0), (0, M - Wcat.shape[1])))
    bcat = jnp.pad(bcat, ((0, 0), (0, M - bcat.shape[1])))
    aw = _attn(herb_feature, Wcat, bcat)

    tgt = jnp.concatenate([p_targets.reshape(-1, 1),
                           n_targets.reshape(-1, 1)], axis=1).astype(jnp.int32)
    b_proj2 = b_proj[None, :]
    b_conf2 = b_conf[None, :]

    c_feat = jnp.pad(ctop, ((0, 0), (0, N2p - N1p), (0, 0)))
    p_feat = n_feat = None
    for o in range(LO):
        for i in range(LI):
            idx = o * LI + i
            aw1 = aw[:, idx * NR:(idx + 1) * NR]
            i_feat = _conv(i_feat, Wc1[idx], aw1, cs1, cd1,
                           src1, dst1, N1p, nc1, kb1)
        aw2 = aw[:, (LI_LO + o) * NR:(LI_LO + o + 1) * NR]
        c_feat = _conv(c_feat, Wc2[o], aw2, cs2, cd2, src2, dst2,
                       N2p, nc2, kb2)
        y2 = _mm_scale(c_feat, Wc2[o], cs2)
        cagg = _make_spmm(N2p, nc2, kb2)(y2.reshape(_B * _NR * N2p, _D),
                                         src2, dst2)
        i_feat, ctop, p_feat, n_feat = _boundary(
            cagg, cd2, aw2, i_feat, W_proj, b_proj2, W_conf, b_conf2,
            W_lin, tgt, N1)
        if o + 1 < LO:
            c_feat = jnp.pad(ctop, ((0, 0), (0, N2p - N1p), (0, 0)))

    return (p_feat[:, 0, :], n_feat[:, 0, :])


# speedup vs baseline: 1.2116x; 1.1343x over previous
import functools

import jax
import jax.numpy as jnp
from jax import lax
from jax.experimental import pallas as pl
from jax.experimental.pallas import tpu as pltpu
from jax.experimental.pallas import tpu_sc as plsc

_B = 2
_D = 128
_NR = 3
_NC = 2
_NS = 16
_K = 128


def _pad_up(n, m):
    return ((n + m - 1) // m) * m


@functools.lru_cache(maxsize=None)
def _make_spmm(N_p, n_chunks, kb):
    rows_pc = N_p // _NS
    mesh = plsc.VectorSubcoreMesh(core_axis_name="c", subcore_axis_name="s",
                                  num_cores=_NC, num_subcores=_NS)

    @functools.partial(
        pl.kernel,
        out_type=jax.ShapeDtypeStruct((_B, _NR, N_p, _D), jnp.float32),
        mesh=mesh,
        scratch_types=[
            pltpu.VMEM_SHARED((N_p, _D), jnp.float32),
            pltpu.VMEM((4, kb), jnp.int32),
            pltpu.VMEM((kb, _D), jnp.float32),
            pltpu.VMEM((kb, _D), jnp.float32),
            pltpu.SemaphoreType.DMA,
            pltpu.SemaphoreType.DMA,
            pltpu.SemaphoreType.DMA,
            pltpu.SemaphoreType.DMA,
        ],
    )
    def spmm(tables, idxc, zin, out, acc, ib, rows0, rows1,
             sem0, sem1, sem2, sem3):
        sid = lax.axis_index("s")
        b = lax.axis_index("c")
        row0 = sid * rows_pc
        for r in range(_NR):
            pltpu.sync_copy(zin.at[pl.ds(row0, rows_pc)],
                            acc.at[pl.ds(row0, rows_pc)])
            plsc.subcore_barrier()

            def ebody(k2, c):
                pltpu.sync_copy(idxc.at[b, r, sid, k2], ib)
                g0 = pltpu.async_copy(tables.at[ib.at[0]], rows0, sem0)
                g1 = pltpu.async_copy(tables.at[ib.at[1]], rows1, sem1)
                g0.wait()
                a0 = pltpu.async_copy(rows0, acc.at[ib.at[2]], sem2, add=True)
                g1.wait()
                a1 = pltpu.async_copy(rows1, acc.at[ib.at[3]], sem3, add=True)
                a0.wait()
                a1.wait()
                return c
            lax.fori_loop(0, n_chunks // 2, ebody, 0)
            plsc.subcore_barrier()
            pltpu.sync_copy(acc.at[pl.ds(row0, rows_pc)],
                            out.at[b, r, pl.ds(row0, rows_pc)])
            plsc.subcore_barrier()

    return spmm


@functools.lru_cache(maxsize=None)
def _make_degrees(N_p, n_chunks):
    rows_pc = N_p // _NS
    mesh = plsc.VectorSubcoreMesh(core_axis_name="c", subcore_axis_name="s",
                                  num_cores=_NC, num_subcores=_NS)

    @functools.partial(
        pl.kernel,
        out_type=jax.ShapeDtypeStruct((2 * _NR, N_p, _D), jnp.float32),
        mesh=mesh,
        scratch_types=[
            pltpu.VMEM_SHARED((N_p, _D), jnp.float32),
            pltpu.VMEM((_K,), jnp.int32),
            pltpu.VMEM((_K, _D), jnp.float32),
            pltpu.VMEM((8, _D), jnp.float32),
            pltpu.VMEM((8, _D), jnp.float32),
        ],
    )
    def deg(ones_tbl, edges_q, out, acc, idxb, ones, zb, ob):
        sid = lax.axis_index("s")
        b = lax.axis_index("c")
        row0 = sid * rows_pc
        zero16 = jnp.zeros((16,), jnp.float32)
        for rr in range(8):
            for cc in range(_D // 16):
                zb[rr, pl.ds(cc * 16, 16)] = zero16
        pltpu.sync_copy(ones_tbl, ones)
        for j3 in range(_NR):
            q = b * _NR + j3
            def zbody(j, c):
                pltpu.sync_copy(zb, acc.at[pl.ds(row0 + j * 8, 8)])
                return c
            lax.fori_loop(0, rows_pc // 8, zbody, 0)
            plsc.subcore_barrier()

            def ebody(k, c):
                pltpu.sync_copy(edges_q.at[q, sid, pl.ds(k * _K, _K)], idxb)
                pltpu.sync_copy(ones, acc.at[idxb], add=True)
                return c
            lax.fori_loop(0, n_chunks, ebody, 0)
            plsc.subcore_barrier()

            def obody(j, c):
                pltpu.sync_copy(acc.at[pl.ds(row0 + j * 8, 8)], ob)
                pltpu.sync_copy(ob, out.at[q, pl.ds(row0 + j * 8, 8)])
                return c
            lax.fori_loop(0, rows_pc // 8, obody, 0)
            plsc.subcore_barrier()

    return deg


def _mm_scale(x, W3, cnt_src):
    B, N_p, D = x.shape
    nblk = N_p // _K

    def body(x_ref, w_ref, c_ref, o_ref):
        do = lax.rsqrt(jnp.maximum(c_ref[0, :, 0:1], 1.0))
        o_ref[0, 0] = jnp.dot(x_ref[0], w_ref[0],
                              preferred_element_type=jnp.float32) * do

    return pl.pallas_call(
        body,
        grid=(B, _NR, nblk),
        in_specs=[
            pl.BlockSpec((1, _K, D), lambda b, r, i: (b, i, 0)),
            pl.BlockSpec((1, D, D), lambda b, r, i: (r, 0, 0)),
            pl.BlockSpec((1, _K, D), lambda b, r, i: (r, i, 0)),
        ],
        out_specs=pl.BlockSpec((1, 1, _K, D), lambda b, r, i: (b, r, i, 0)),
        out_shape=jax.ShapeDtypeStruct((B, _NR, N_p, D), jnp.float32),
    )(x, W3, cnt_src)


def _combine(agg, cnt_dst, aw):
    B, NR, N_p, D = agg.shape
    nblk = N_p // _K

    def body(a_ref, c_ref, aw_ref, o_ref):
        b = pl.program_id(0)
        acc = jnp.zeros((_K, D), jnp.float32)
        for r in range(_NR):
            di = lax.rsqrt(jnp.maximum(c_ref[r, :, 0:1], 1.0))
            acc = acc + jax.nn.relu(a_ref[0, r] * (di * aw_ref[b, r]))
        o_ref[0] = acc

    return pl.pallas_call(
        body,
        grid=(B, nblk),
        in_specs=[
            pl.BlockSpec((1, _NR, _K, D), lambda b, i: (b, 0, i, 0)),
            pl.BlockSpec((_NR, _K, D), lambda b, i: (0, i, 0)),
            pl.BlockSpec(memory_space=pltpu.SMEM),
        ],
        out_specs=pl.BlockSpec((1, _K, D), lambda b, i: (b, i, 0)),
        out_shape=jax.ShapeDtypeStruct((B, N_p, D), jnp.float32),
    )(agg, cnt_dst, aw)


def _boundary(cagg, cnt_dst, aw, i_feat, W_proj, b_proj2, W_conf, b_conf2,
              W_lin, tgt, N1):
    B, NR, N2p, D = cagg.shape
    N1p = i_feat.shape[1]
    nblk = N1p // _K

    def body(a_ref, c_ref, aw_ref, if_ref, wp_ref, bp_ref, wc_ref, bc_ref,
             wl_ref, t_ref, f_ref, ct_ref, p_ref, n_ref):
        b = pl.program_id(0)
        i = pl.program_id(1)
        acc = jnp.zeros((_K, D), jnp.float32)
        for r in range(_NR):
            di = lax.rsqrt(jnp.maximum(c_ref[r, :, 0:1], 1.0))
            acc = acc + jax.nn.relu(a_ref[0, r] * (di * aw_ref[b, r]))
        proj = jnp.dot(acc, wp_ref[...], preferred_element_type=jnp.float32)
        proj = proj + bp_ref[0]
        conf = jax.nn.sigmoid(
            jnp.dot(acc, wc_ref[...], preferred_element_type=jnp.float32)
            + bc_ref[0])
        f = conf * (if_ref[0] + proj)
        rows = i * _K + lax.broadcasted_iota(jnp.int32, (_K, 1), 0)
        f = f * (rows < N1).astype(jnp.float32)
        f_ref[0] = f
        ct_ref[0] = jnp.dot(f, wl_ref[...], preferred_element_type=jnp.float32)
        psel = jnp.sum(jnp.where(rows == t_ref[b, 0], f, 0.0), axis=0)
        nsel = jnp.sum(jnp.where(rows == t_ref[b, 1], f, 0.0), axis=0)
        psel = jnp.broadcast_to(psel[None, :], (8, f.shape[1]))
        nsel = jnp.broadcast_to(nsel[None, :], (8, f.shape[1]))
        first = (i == 0)
        p_ref[0] = jnp.where(first, psel, p_ref[0] + psel)
        n_ref[0] = jnp.where(first, nsel, n_ref[0] + nsel)

    return pl.pallas_call(
        body,
        grid=(B, nblk),
        in_specs=[
            pl.BlockSpec((1, _NR, _K, D), lambda b, i: (b, 0, i, 0)),
            pl.BlockSpec((_NR, _K, D), lambda b, i: (0, i, 0)),
            pl.BlockSpec(memory_space=pltpu.SMEM),
            pl.BlockSpec((1, _K, D), lambda b, i: (b, i, 0)),
            pl.BlockSpec((D, D), lambda b, i: (0, 0)),
            pl.BlockSpec((1, D), lambda b, i: (0, 0)),
            pl.BlockSpec((D, D), lambda b, i: (0, 0)),
            pl.BlockSpec((1, D), lambda b, i: (0, 0)),
            pl.BlockSpec((D, D), lambda b, i: (0, 0)),
            pl.BlockSpec(memory_space=pltpu.SMEM),
        ],
        out_specs=[
            pl.BlockSpec((1, _K, D), lambda b, i: (b, i, 0)),
            pl.BlockSpec((1, _K, D), lambda b, i: (b, i, 0)),
            pl.BlockSpec((1, 8, D), lambda b, i: (b, 0, 0)),
            pl.BlockSpec((1, 8, D), lambda b, i: (b, 0, 0)),
        ],
        out_shape=[
            jax.ShapeDtypeStruct((B, N1p, D), jnp.float32),
            jax.ShapeDtypeStruct((B, N1p, D), jnp.float32),
            jax.ShapeDtypeStruct((B, 8, D), jnp.float32),
            jax.ShapeDtypeStruct((B, 8, D), jnp.float32),
        ],
    )(cagg, cnt_dst, aw, i_feat, W_proj, b_proj2, W_conf, b_conf2, W_lin, tgt)


def _seed(hof3, tm, herb3, W_lin):
    B = hof3.shape[0]
    NHp = hof3.shape[2]
    N1p = tm.shape[1]
    D = herb3.shape[2]
    nblk = N1p // _K

    def body(m_ref, tm_ref, h_ref, wl_ref, i0_ref, c0_ref):
        m = (m_ref[0] > 0).astype(jnp.float32)
        s = jnp.dot(m, tm_ref[...], preferred_element_type=jnp.float32)
        h = h_ref[0]
        hw = jnp.dot(h, wl_ref[...], preferred_element_type=jnp.float32)
        i0_ref[0] = s[0][:, None] * h[0][None, :]
        c0_ref[0] = s[0][:, None] * hw[0][None, :]

    return pl.pallas_call(
        body,
        grid=(B, nblk),
        in_specs=[
            pl.BlockSpec((1, 1, NHp), lambda b, i: (b, 0, 0)),
            pl.BlockSpec((NHp, _K), lambda b, i: (0, i)),
            pl.BlockSpec((1, 1, D), lambda b, i: (b, 0, 0)),
            pl.BlockSpec((D, D), lambda b, i: (0, 0)),
        ],
        out_specs=[
            pl.BlockSpec((1, _K, D), lambda b, i: (b, i, 0)),
            pl.BlockSpec((1, _K, D), lambda b, i: (b, i, 0)),
        ],
        out_shape=[
            jax.ShapeDtypeStruct((B, N1p, D), jnp.float32),
            jax.ShapeDtypeStruct((B, N1p, D), jnp.float32),
        ],
    )(hof3, tm, herb3, W_lin)


def _attn(herb, Wcat, bcat):
    B, D = herb.shape
    M = Wcat.shape[1]

    def body(x_ref, w_ref, b_ref, o_ref):
        o_ref[...] = jax.nn.sigmoid(
            jnp.dot(x_ref[...], w_ref[...],
                    preferred_element_type=jnp.float32) + b_ref[0])

    return pl.pallas_call(
        body,
        grid=(1,),
        in_specs=[
            pl.BlockSpec((B, D), lambda k: (0, 0)),
            pl.BlockSpec((D, M), lambda k: (0, 0)),
            pl.BlockSpec((1, M), lambda k: (0, 0)),
        ],
        out_specs=pl.BlockSpec((B, M), lambda k: (0, 0)),
        out_shape=jax.ShapeDtypeStruct((B, M), jnp.float32),
    )(herb, Wcat, bcat)


def _prep_edges(edges, N, N_p, kb):
    E = edges.shape[2]
    epc = E // _NS
    epc_p = _pad_up(epc, max(2 * kb, _K))
    n_chunks = epc_p // kb
    pad = epc_p - epc
    e = edges.astype(jnp.int32).reshape(_NR, 2, _NS, epc)
    src = jnp.pad(e[:, 0], ((0, 0), (0, 0), (0, pad)))
    dst = jnp.pad(e[:, 1], ((0, 0), (0, 0), (0, pad)),
                  constant_values=N)
    roff = (jnp.arange(_NR, dtype=jnp.int32) * N_p)[None, :, None, None]
    boff = (jnp.arange(_B, dtype=jnp.int32) * (_NR * N_p))[:, None, None, None]
    srcp = (src[None] + roff + boff).reshape(_B, _NR, _NS, n_chunks // 2,
                                             2, kb)
    dstp = jnp.broadcast_to(dst.reshape(_NR, _NS, n_chunks // 2, 2, kb)[None],
                            srcp.shape)
    idxc = jnp.concatenate([srcp, dstp], axis=4)
    src_q = jnp.pad(e[:, 0], ((0, 0), (0, 0), (0, pad)), constant_values=N)
    edges_q = jnp.stack([src_q, dst], axis=1).reshape(2 * _NR, _NS, epc_p)
    return idxc, edges_q, n_chunks, epc_p // _K


def _counts(edges_q, N_p, n_chunks):
    ones_tbl = jnp.ones((_K, _D), jnp.float32)
    cnt = _make_degrees(N_p, n_chunks)(ones_tbl, edges_q)
    return cnt[0::2], cnt[1::2]


def _conv(x, W3, aw, cs, cd, idxc, zin, N_p, n_chunks, kb):
    y = _mm_scale(x, W3, cs)
    agg = _make_spmm(N_p, n_chunks, kb)(y.reshape(_B * _NR * N_p, _D),
                                        idxc, zin)
    return _combine(agg, cd, aw)


def kernel(herb_feature, herb_original_feat, graph1_edges, graph2_edges,
           p_targets, n_targets, total_map, W_lin, Wa1, ba1, Wa2, ba2,
           Wc1, Wc2, W_proj, b_proj, W_conf, b_conf):
    B, NH = herb_original_feat.shape
    N1 = total_map.shape[1]
    N2 = int(graph2_edges.shape[2] // 16)
    LI_LO = Wa1.shape[0]
    LO = Wa2.shape[0]
    LI = LI_LO // LO
    NR = Wa1.shape[2]
    N1p = _pad_up(N1, _NS * 8)
    N2p = _pad_up(N2, _NS * 8)
    NHp = _pad_up(NH, 512)

    kb1 = 128 if N1p <= 8192 else 64
    kb2 = 128 if N2p <= 8192 else 64
    idx1, eq1, nc1, ncq1 = _prep_edges(graph1_edges, N1, N1p, kb1)
    idx2, eq2, nc2, ncq2 = _prep_edges(graph2_edges, N2, N2p, kb2)
    zin1 = jnp.zeros((N1p, _D), jnp.float32)
    zin2 = jnp.zeros((N2p, _D), jnp.float32)
    cs1, cd1 = _counts(eq1, N1p, ncq1)
    cs2, cd2 = _counts(eq2, N2p, ncq2)

    hof3 = jnp.pad(herb_original_feat.astype(jnp.int32),
                   ((0, 0), (0, NHp - NH)))[:, None, :]
    tmp = jnp.pad(total_map, ((0, NHp - NH), (0, N1p - N1)))
    herb3 = herb_feature[:, None, :]
    i_feat, ctop = _seed(hof3, tmp, herb3, W_lin)

    Wcat = jnp.concatenate(list(Wa1) + list(Wa2), axis=1)
    bcat = jnp.concatenate(list(ba1) + list(ba2), axis=0)[None, :]
    M = _pad_up(Wcat.shape[1], 128)
    Wcat = jnp.pad(Wcat, ((0, 0), (0, M - Wcat.shape[1])))
    bcat = jnp.pad(bcat, ((0, 0), (0, M - bcat.shape[1])))
    aw = _attn(herb_feature, Wcat, bcat)

    tgt = jnp.concatenate([p_targets.reshape(-1, 1),
                           n_targets.reshape(-1, 1)], axis=1).astype(jnp.int32)
    b_proj2 = b_proj[None, :]
    b_conf2 = b_conf[None, :]

    c_feat = jnp.pad(ctop, ((0, 0), (0, N2p - N1p), (0, 0)))
    p_feat = n_feat = None
    for o in range(LO):
        for i in range(LI):
            idx = o * LI + i
            aw1 = aw[:, idx * NR:(idx + 1) * NR]
            i_feat = _conv(i_feat, Wc1[idx], aw1, cs1, cd1,
                           idx1, zin1, N1p, nc1, kb1)
        aw2 = aw[:, (LI_LO + o) * NR:(LI_LO + o + 1) * NR]
        c_feat = _conv(c_feat, Wc2[o], aw2, cs2, cd2, idx2, zin2,
                       N2p, nc2, kb2)
        y2 = _mm_scale(c_feat, Wc2[o], cs2)
        cagg = _make_spmm(N2p, nc2, kb2)(y2.reshape(_B * _NR * N2p, _D),
                                         idx2, zin2)
        i_feat, ctop, p_feat, n_feat = _boundary(
            cagg, cd2, aw2, i_feat, W_proj, b_proj2, W_conf, b_conf2,
            W_lin, tgt, N1)
        if o + 1 < LO:
            c_feat = jnp.pad(ctop, ((0, 0), (0, N2p - N1p), (0, 0)))

    return (p_feat[:, 0, :], n_feat[:, 0, :])


# speedup vs baseline: 1.2128x; 1.0010x over previous
import functools

import jax
import jax.numpy as jnp
from jax import lax
from jax.experimental import pallas as pl
from jax.experimental.pallas import tpu as pltpu
from jax.experimental.pallas import tpu_sc as plsc

_B = 2
_D = 128
_NR = 3
_NC = 2
_NS = 16
_K = 128


def _pad_up(n, m):
    return ((n + m - 1) // m) * m


@functools.lru_cache(maxsize=None)
def _make_spmm(N_p, n_chunks, kb):
    rows_pc = N_p // _NS
    mesh = plsc.VectorSubcoreMesh(core_axis_name="c", subcore_axis_name="s",
                                  num_cores=_NC, num_subcores=_NS)

    @functools.partial(
        pl.kernel,
        out_type=jax.ShapeDtypeStruct((_B, _NR, N_p, _D), jnp.float32),
        mesh=mesh,
        scratch_types=[
            pltpu.VMEM_SHARED((N_p, _D), jnp.float32),
            pltpu.VMEM((4, kb), jnp.int32),
            pltpu.VMEM((kb, _D), jnp.float32),
            pltpu.VMEM((kb, _D), jnp.float32),
            pltpu.SemaphoreType.DMA,
            pltpu.SemaphoreType.DMA,
            pltpu.SemaphoreType.DMA,
            pltpu.SemaphoreType.DMA,
        ],
    )
    def spmm(tables, idxc, zin, out, acc, ib, rows0, rows1,
             sem0, sem1, sem2, sem3):
        sid = lax.axis_index("s")
        b = lax.axis_index("c")
        row0 = sid * rows_pc
        for r in range(_NR):
            pltpu.sync_copy(zin.at[pl.ds(row0, rows_pc)],
                            acc.at[pl.ds(row0, rows_pc)])
            plsc.subcore_barrier()

            def ebody(k2, c):
                pltpu.sync_copy(idxc.at[b, r, sid, k2], ib)
                g0 = pltpu.async_copy(tables.at[ib.at[0]], rows0, sem0)
                g1 = pltpu.async_copy(tables.at[ib.at[1]], rows1, sem1)
                g0.wait()
                a0 = pltpu.async_copy(rows0, acc.at[ib.at[2]], sem2, add=True)
                g1.wait()
                a1 = pltpu.async_copy(rows1, acc.at[ib.at[3]], sem3, add=True)
                a0.wait()
                a1.wait()
                return c
            lax.fori_loop(0, n_chunks // 2, ebody, 0)
            plsc.subcore_barrier()
            pltpu.sync_copy(acc.at[pl.ds(row0, rows_pc)],
                            out.at[b, r, pl.ds(row0, rows_pc)])
            plsc.subcore_barrier()

    return spmm


@functools.lru_cache(maxsize=None)
def _make_degrees(N_p, n_chunks):
    rows_pc = N_p // _NS
    mesh = plsc.VectorSubcoreMesh(core_axis_name="c", subcore_axis_name="s",
                                  num_cores=_NC, num_subcores=_NS)

    @functools.partial(
        pl.kernel,
        out_type=jax.ShapeDtypeStruct((2 * _NR, N_p, _D), jnp.float32),
        mesh=mesh,
        scratch_types=[
            pltpu.VMEM_SHARED((N_p, _D), jnp.float32),
            pltpu.VMEM((_K,), jnp.int32),
            pltpu.VMEM((_K, _D), jnp.float32),
            pltpu.VMEM((8, _D), jnp.float32),
            pltpu.VMEM((8, _D), jnp.float32),
        ],
    )
    def deg(ones_tbl, edges_q, out, acc, idxb, ones, zb, ob):
        sid = lax.axis_index("s")
        b = lax.axis_index("c")
        row0 = sid * rows_pc
        zero16 = jnp.zeros((16,), jnp.float32)
        for rr in range(8):
            for cc in range(_D // 16):
                zb[rr, pl.ds(cc * 16, 16)] = zero16
        pltpu.sync_copy(ones_tbl, ones)
        for j3 in range(_NR):
            q = b * _NR + j3
            def zbody(j, c):
                pltpu.sync_copy(zb, acc.at[pl.ds(row0 + j * 8, 8)])
                return c
            lax.fori_loop(0, rows_pc // 8, zbody, 0)
            plsc.subcore_barrier()

            def ebody(k, c):
                pltpu.sync_copy(edges_q.at[q, sid, pl.ds(k * _K, _K)], idxb)
                pltpu.sync_copy(ones, acc.at[idxb], add=True)
                return c
            lax.fori_loop(0, n_chunks, ebody, 0)
            plsc.subcore_barrier()

            def obody(j, c):
                pltpu.sync_copy(acc.at[pl.ds(row0 + j * 8, 8)], ob)
                pltpu.sync_copy(ob, out.at[q, pl.ds(row0 + j * 8, 8)])
                return c
            lax.fori_loop(0, rows_pc // 8, obody, 0)
            plsc.subcore_barrier()

    return deg


def _mm_scale(x, W3, cnt_src):
    B, N_p, D = x.shape
    nblk = N_p // _K

    def body(x_ref, w_ref, c_ref, o_ref):
        do = lax.rsqrt(jnp.maximum(c_ref[0, :, 0:1], 1.0))
        o_ref[0, 0] = jnp.dot(x_ref[0], w_ref[0],
                              preferred_element_type=jnp.float32) * do

    return pl.pallas_call(
        body,
        grid=(B, _NR, nblk),
        in_specs=[
            pl.BlockSpec((1, _K, D), lambda b, r, i: (b, i, 0)),
            pl.BlockSpec((1, D, D), lambda b, r, i: (r, 0, 0)),
            pl.BlockSpec((1, _K, D), lambda b, r, i: (r, i, 0)),
        ],
        out_specs=pl.BlockSpec((1, 1, _K, D), lambda b, r, i: (b, r, i, 0)),
        out_shape=jax.ShapeDtypeStruct((B, _NR, N_p, D), jnp.float32),
    )(x, W3, cnt_src)


def _combine(agg, cnt_dst, aw):
    B, NR, N_p, D = agg.shape
    nblk = N_p // _K

    def body(a_ref, c_ref, aw_ref, o_ref):
        b = pl.program_id(0)
        acc = jnp.zeros((_K, D), jnp.float32)
        for r in range(_NR):
            di = lax.rsqrt(jnp.maximum(c_ref[r, :, 0:1], 1.0))
            acc = acc + jax.nn.relu(a_ref[0, r] * (di * aw_ref[b, r]))
        o_ref[0] = acc

    return pl.pallas_call(
        body,
        grid=(B, nblk),
        in_specs=[
            pl.BlockSpec((1, _NR, _K, D), lambda b, i: (b, 0, i, 0)),
            pl.BlockSpec((_NR, _K, D), lambda b, i: (0, i, 0)),
            pl.BlockSpec(memory_space=pltpu.SMEM),
        ],
        out_specs=pl.BlockSpec((1, _K, D), lambda b, i: (b, i, 0)),
        out_shape=jax.ShapeDtypeStruct((B, N_p, D), jnp.float32),
    )(agg, cnt_dst, aw)


def _boundary(cagg, cnt_dst, aw, i_feat, W_proj, b_proj2, W_conf, b_conf2,
              W_lin, tgt, N1):
    B, NR, N2p, D = cagg.shape
    N1p = i_feat.shape[1]
    nblk = N1p // _K

    def body(a_ref, c_ref, aw_ref, if_ref, wp_ref, bp_ref, wc_ref, bc_ref,
             wl_ref, t_ref, f_ref, ct_ref, p_ref, n_ref):
        b = pl.program_id(0)
        i = pl.program_id(1)
        acc = jnp.zeros((_K, D), jnp.float32)
        for r in range(_NR):
            di = lax.rsqrt(jnp.maximum(c_ref[r, :, 0:1], 1.0))
            acc = acc + jax.nn.relu(a_ref[0, r] * (di * aw_ref[b, r]))
        proj = jnp.dot(acc, wp_ref[...], preferred_element_type=jnp.float32)
        proj = proj + bp_ref[0]
        conf = jax.nn.sigmoid(
            jnp.dot(acc, wc_ref[...], preferred_element_type=jnp.float32)
            + bc_ref[0])
        f = conf * (if_ref[0] + proj)
        rows = i * _K + lax.broadcasted_iota(jnp.int32, (_K, 1), 0)
        f = f * (rows < N1).astype(jnp.float32)
        f_ref[0] = f
        ct_ref[0] = jnp.dot(f, wl_ref[...], preferred_element_type=jnp.float32)
        psel = jnp.sum(jnp.where(rows == t_ref[b, 0], f, 0.0), axis=0)
        nsel = jnp.sum(jnp.where(rows == t_ref[b, 1], f, 0.0), axis=0)
        psel = jnp.broadcast_to(psel[None, :], (8, f.shape[1]))
        nsel = jnp.broadcast_to(nsel[None, :], (8, f.shape[1]))
        first = (i == 0)
        p_ref[0] = jnp.where(first, psel, p_ref[0] + psel)
        n_ref[0] = jnp.where(first, nsel, n_ref[0] + nsel)

    return pl.pallas_call(
        body,
        grid=(B, nblk),
        in_specs=[
            pl.BlockSpec((1, _NR, _K, D), lambda b, i: (b, 0, i, 0)),
            pl.BlockSpec((_NR, _K, D), lambda b, i: (0, i, 0)),
            pl.BlockSpec(memory_space=pltpu.SMEM),
            pl.BlockSpec((1, _K, D), lambda b, i: (b, i, 0)),
            pl.BlockSpec((D, D), lambda b, i: (0, 0)),
            pl.BlockSpec((1, D), lambda b, i: (0, 0)),
            pl.BlockSpec((D, D), lambda b, i: (0, 0)),
            pl.BlockSpec((1, D), lambda b, i: (0, 0)),
            pl.BlockSpec((D, D), lambda b, i: (0, 0)),
            pl.BlockSpec(memory_space=pltpu.SMEM),
        ],
        out_specs=[
            pl.BlockSpec((1, _K, D), lambda b, i: (b, i, 0)),
            pl.BlockSpec((1, _K, D), lambda b, i: (b, i, 0)),
            pl.BlockSpec((1, 8, D), lambda b, i: (b, 0, 0)),
            pl.BlockSpec((1, 8, D), lambda b, i: (b, 0, 0)),
        ],
        out_shape=[
            jax.ShapeDtypeStruct((B, N1p, D), jnp.float32),
            jax.ShapeDtypeStruct((B, N1p, D), jnp.float32),
            jax.ShapeDtypeStruct((B, 8, D), jnp.float32),
            jax.ShapeDtypeStruct((B, 8, D), jnp.float32),
        ],
    )(cagg, cnt_dst, aw, i_feat, W_proj, b_proj2, W_conf, b_conf2, W_lin, tgt)


def _seed(hof3, tm, herb3, W_lin):
    B = hof3.shape[0]
    NHp = hof3.shape[2]
    N1p = tm.shape[1]
    D = herb3.shape[2]
    nblk = N1p // _K

    def body(m_ref, tm_ref, h_ref, wl_ref, i0_ref, c0_ref):
        m = (m_ref[0] > 0).astype(jnp.float32)
        s = jnp.dot(m, tm_ref[...], preferred_element_type=jnp.float32)
        h = h_ref[0]
        hw = jnp.dot(h, wl_ref[...], preferred_element_type=jnp.float32)
        i0_ref[0] = s[0][:, None] * h[0][None, :]
        c0_ref[0] = s[0][:, None] * hw[0][None, :]

    return pl.pallas_call(
        body,
        grid=(B, nblk),
        in_specs=[
            pl.BlockSpec((1, 1, NHp), lambda b, i: (b, 0, 0)),
            pl.BlockSpec((NHp, _K), lambda b, i: (0, i)),
            pl.BlockSpec((1, 1, D), lambda b, i: (b, 0, 0)),
            pl.BlockSpec((D, D), lambda b, i: (0, 0)),
        ],
        out_specs=[
            pl.BlockSpec((1, _K, D), lambda b, i: (b, i, 0)),
            pl.BlockSpec((1, _K, D), lambda b, i: (b, i, 0)),
        ],
        out_shape=[
            jax.ShapeDtypeStruct((B, N1p, D), jnp.float32),
            jax.ShapeDtypeStruct((B, N1p, D), jnp.float32),
        ],
    )(hof3, tm, herb3, W_lin)


def _attn(herb, Wcat, bcat):
    B, D = herb.shape
    M = Wcat.shape[1]

    def body(x_ref, w_ref, b_ref, o_ref):
        o_ref[...] = jax.nn.sigmoid(
            jnp.dot(x_ref[...], w_ref[...],
                    preferred_element_type=jnp.float32) + b_ref[0])

    return pl.pallas_call(
        body,
        grid=(1,),
        in_specs=[
            pl.BlockSpec((B, D), lambda k: (0, 0)),
            pl.BlockSpec((D, M), lambda k: (0, 0)),
            pl.BlockSpec((1, M), lambda k: (0, 0)),
        ],
        out_specs=pl.BlockSpec((B, M), lambda k: (0, 0)),
        out_shape=jax.ShapeDtypeStruct((B, M), jnp.float32),
    )(herb, Wcat, bcat)


def _prep_edges(edges, N, N_p, kb):
    E = edges.shape[2]
    epc = E // _NS
    epc_p = _pad_up(epc, max(2 * kb, _K))
    n_chunks = epc_p // kb
    pad = epc_p - epc
    e = edges.astype(jnp.int32).reshape(_NR, 2, _NS, epc)
    src = jnp.pad(e[:, 0], ((0, 0), (0, 0), (0, pad)))
    dst = jnp.pad(e[:, 1], ((0, 0), (0, 0), (0, pad)),
                  constant_values=N)
    roff = (jnp.arange(_NR, dtype=jnp.int32) * N_p)[None, :, None, None]
    boff = (jnp.arange(_B, dtype=jnp.int32) * (_NR * N_p))[:, None, None, None]
    srcp = (src[None] + roff + boff).reshape(_B, _NR, _NS, n_chunks // 2,
                                             2, kb)
    dstp = jnp.broadcast_to(dst.reshape(_NR, _NS, n_chunks // 2, 2, kb)[None],
                            srcp.shape)
    idxc = jnp.concatenate([srcp, dstp], axis=4)
    src_q = jnp.pad(e[:, 0], ((0, 0), (0, 0), (0, pad)), constant_values=N)
    edges_q = jnp.stack([src_q, dst], axis=1).reshape(2 * _NR, _NS, epc_p)
    return idxc, edges_q, n_chunks, epc_p // _K


def _counts(edges_q, N_p, n_chunks):
    ones_tbl = jnp.ones((_K, _D), jnp.float32)
    cnt = _make_degrees(N_p, n_chunks)(ones_tbl, edges_q)
    return cnt[0::2], cnt[1::2]


def _conv(x, W3, aw, cs, cd, idxc, zin, N_p, n_chunks, kb):
    y = _mm_scale(x, W3, cs)
    agg = _make_spmm(N_p, n_chunks, kb)(y.reshape(_B * _NR * N_p, _D),
                                        idxc, zin)
    return _combine(agg, cd, aw)


def kernel(herb_feature, herb_original_feat, graph1_edges, graph2_edges,
           p_targets, n_targets, total_map, W_lin, Wa1, ba1, Wa2, ba2,
           Wc1, Wc2, W_proj, b_proj, W_conf, b_conf):
    B, NH = herb_original_feat.shape
    N1 = total_map.shape[1]
    N2 = int(graph2_edges.shape[2] // 16)
    LI_LO = Wa1.shape[0]
    LO = Wa2.shape[0]
    LI = LI_LO // LO
    NR = Wa1.shape[2]
    N1p = _pad_up(N1, _NS * 8)
    N2p = _pad_up(N2, _NS * 8)
    NHp = _pad_up(NH, 512)

    kb1 = 128 if N1p <= 8192 else 80
    kb2 = 128 if N2p <= 8192 else 80
    idx1, eq1, nc1, ncq1 = _prep_edges(graph1_edges, N1, N1p, kb1)
    idx2, eq2, nc2, ncq2 = _prep_edges(graph2_edges, N2, N2p, kb2)
    zin1 = jnp.zeros((N1p, _D), jnp.float32)
    zin2 = jnp.zeros((N2p, _D), jnp.float32)
    cs1, cd1 = _counts(eq1, N1p, ncq1)
    cs2, cd2 = _counts(eq2, N2p, ncq2)

    hof3 = jnp.pad(herb_original_feat.astype(jnp.int32),
                   ((0, 0), (0, NHp - NH)))[:, None, :]
    tmp = jnp.pad(total_map, ((0, NHp - NH), (0, N1p - N1)))
    herb3 = herb_feature[:, None, :]
    i_feat, ctop = _seed(hof3, tmp, herb3, W_lin)

    Wcat = jnp.concatenate(list(Wa1) + list(Wa2), axis=1)
    bcat = jnp.concatenate(list(ba1) + list(ba2), axis=0)[None, :]
    M = _pad_up(Wcat.shape[1], 128)
    Wcat = jnp.pad(Wcat, ((0, 0), (0, M - Wcat.shape[1])))
    bcat = jnp.pad(bcat, ((0, 0), (0, M - bcat.shape[1])))
    aw = _attn(herb_feature, Wcat, bcat)

    tgt = jnp.concatenate([p_targets.reshape(-1, 1),
                           n_targets.reshape(-1, 1)], axis=1).astype(jnp.int32)
    b_proj2 = b_proj[None, :]
    b_conf2 = b_conf[None, :]

    c_feat = jnp.pad(ctop, ((0, 0), (0, N2p - N1p), (0, 0)))
    p_feat = n_feat = None
    for o in range(LO):
        for i in range(LI):
            idx = o * LI + i
            aw1 = aw[:, idx * NR:(idx + 1) * NR]
            i_feat = _conv(i_feat, Wc1[idx], aw1, cs1, cd1,
                           idx1, zin1, N1p, nc1, kb1)
        aw2 = aw[:, (LI_LO + o) * NR:(LI_LO + o + 1) * NR]
        c_feat = _conv(c_feat, Wc2[o], aw2, cs2, cd2, idx2, zin2,
                       N2p, nc2, kb2)
        y2 = _mm_scale(c_feat, Wc2[o], cs2)
        cagg = _make_spmm(N2p, nc2, kb2)(y2.reshape(_B * _NR * N2p, _D),
                                         idx2, zin2)
        i_feat, ctop, p_feat, n_feat = _boundary(
            cagg, cd2, aw2, i_feat, W_proj, b_proj2, W_conf, b_conf2,
            W_lin, tgt, N1)
        if o + 1 < LO:
            c_feat = jnp.pad(ctop, ((0, 0), (0, N2p - N1p), (0, 0)))

    return (p_feat[:, 0, :], n_feat[:, 0, :])
